# Initial kernel scaffold; baseline (speedup 1.0000x reference)
import jax
import jax.numpy as jnp
from jax.experimental import pallas as pl


def kernel(x, Wq, bq, Wk, bk, Wv, bv, Wo, bo, log_sigma, current_pos):
    raise NotImplementedError("write your pallas kernel here")


# speedup vs baseline: 32.9227x; 32.9227x over previous
import jax
import jax.numpy as jnp
from jax.experimental import pallas as pl
from jax.experimental.pallas import tpu as pltpu

_TOPK = 96


def _krause_kernel(x_ref, wq_ref, wk_ref, wv_ref, wo_ref,
                   bq_ref, bk_ref, bv_ref, bo_ref, ls_ref, out_ref):
    h = pl.program_id(0)
    x = x_ref[...]
    dn = (((1,), (1,)), ((), ()))
    q = jax.lax.dot_general(x, wq_ref[...], dn,
                            preferred_element_type=jnp.float32) + bq_ref[0]
    k = jax.lax.dot_general(x, wk_ref[...], dn,
                            preferred_element_type=jnp.float32) + bk_ref[0]
    v = jax.lax.dot_general(x, wv_ref[...], dn,
                            preferred_element_type=jnp.float32) + bv_ref[0]

    d_real = jnp.sum((q - k) ** 2, axis=1, keepdims=True)
    d_zero = jnp.sum(q * q, axis=1, keepdims=True)
    ls = ls_ref[0, 0, 0]
    inv_two_sigma_sq = 0.5 * jnp.exp(-2.0 * ls)
    z = (d_real - d_zero) * inv_two_sigma_sq
    gate = jnp.where(d_real < d_zero,
                     1.0 / (1.0 + (_TOPK - 1) * jnp.exp(z)),
                     0.0)

    y = v * gate
    partial = jax.lax.dot_general(y, wo_ref[...], dn,
                                  preferred_element_type=jnp.float32)

    @pl.when(h == 0)
    def _init():
        out_ref[...] = partial + bo_ref[...]

    @pl.when(h != 0)
    def _acc():
        out_ref[...] += partial


def kernel(x, Wq, bq, Wk, bk, Wv, bv, Wo, bo, log_sigma, current_pos):
    del current_pos
    B, T, E = x.shape
    H = log_sigma.shape[0]
    DH = E // H

    xf = x.reshape(B, E)
    bq2 = bq.reshape(H, 1, DH)
    bk2 = bk.reshape(H, 1, DH)
    bv2 = bv.reshape(H, 1, DH)
    bo2 = bo.reshape(1, E)
    ls2 = log_sigma.reshape(H, 1, 1)

    out = pl.pallas_call(
        _krause_kernel,
        grid=(H,),
        in_specs=[
            pl.BlockSpec((B, E), lambda h: (0, 0)),
            pl.BlockSpec((DH, E), lambda h: (h, 0)),
            pl.BlockSpec((DH, E), lambda h: (h, 0)),
            pl.BlockSpec((DH, E), lambda h: (h, 0)),
            pl.BlockSpec((E, DH), lambda h: (0, h)),
            pl.BlockSpec((1, 1, DH), lambda h: (h, 0, 0)),
            pl.BlockSpec((1, 1, DH), lambda h: (h, 0, 0)),
            pl.BlockSpec((1, 1, DH), lambda h: (h, 0, 0)),
            pl.BlockSpec((1, E), lambda h: (0, 0)),
            pl.BlockSpec((1, 1, 1), lambda h: (h, 0, 0)),
        ],
        out_specs=pl.BlockSpec((B, E), lambda h: (0, 0)),
        out_shape=jax.ShapeDtypeStruct((B, E), jnp.float32),
        compiler_params=pltpu.CompilerParams(
            dimension_semantics=("arbitrary",)),
    )(xf, Wq, Wk, Wv, Wo, bq2, bk2, bv2, bo2, ls2)

    return out.reshape(B, 1, E)
